# baseline (speedup 1.0000x reference)
import jax
import jax.numpy as jnp
from jax.experimental import pallas as pl
from jax.experimental.pallas import tpu as pltpu

N = 4096
D = 256
DL = 16
H = 256
N_HEMI = 2048
TOPK = 20

BLOCK_ROWS = [128, 512, 512, 512, 384] + [384, 512, 512, 512, 128]
BLOCK_OFFS = [sum(BLOCK_ROWS[:i]) for i in range(len(BLOCK_ROWS))]
N_BLKS = len(BLOCK_ROWS)
HEMI_BLKS = N_BLKS // 2
MAX_ROWS = max(BLOCK_ROWS)
NBUF = 3


def _topk_sum_and_max(x):
    r, c = x.shape
    flat = (
        jax.lax.broadcasted_iota(jnp.int32, (r, c), 0) * c
        + jax.lax.broadcasted_iota(jnp.int32, (r, c), 1)
    )
    big = jnp.int32(2**30)
    neg_inf = jnp.float32(-jnp.inf)
    cur = x
    acc = jnp.float32(0.0)
    mx = neg_inf
    for _ in range(TOPK):
        m = jnp.max(cur)
        idx = jnp.min(jnp.where(cur == m, flat, big))
        cur = jnp.where(flat == idx, neg_inf, cur)
        acc = acc + m
        mx = jnp.maximum(mx, m)
    return acc, mx


def _fused_kernel(x_hbm, adj_hbm, lloc_ref, w1_ref, wl_ref, w2_ref, b2_ref,
                  wc_ref, bc_ref, out_ref, x_vmem, bufs, sems, xsem):
    def copy_blk(b):
        return pltpu.make_async_copy(
            adj_hbm.at[pl.ds(BLOCK_OFFS[b], BLOCK_ROWS[b]), :],
            bufs.at[b % NBUF, pl.ds(0, BLOCK_ROWS[b])],
            sems.at[b % NBUF])

    pltpu.make_async_copy(x_hbm, x_vmem, xsem).start()
    for b in range(min(NBUF, N_BLKS)):
        copy_blk(b).start()

    pltpu.make_async_copy(x_hbm, x_vmem, xsem).wait()
    u = jnp.dot(x_vmem[...], w1_ref[...], preferred_element_type=jnp.float32)
    w2d = w2_ref[:, 1:2] - w2_ref[:, 0:1]
    b2d = b2_ref[0, 1] - b2_ref[0, 0]

    dparts = []
    heads = []
    for b in range(N_BLKS):
        rows = BLOCK_ROWS[b]
        copy_blk(b).wait()
        m = jnp.dot(bufs[b % NBUF, 0:rows], u,
                    preferred_element_type=jnp.float32)
        m = m + jnp.dot(lloc_ref[pl.ds(BLOCK_OFFS[b], rows), :], wl_ref[...],
                        preferred_element_type=jnp.float32)
        dblk = jnp.dot(jax.nn.relu(m), w2d,
                       preferred_element_type=jnp.float32) + b2d
        dparts.append(dblk.reshape(rows // 128, 128))
        if b + NBUF < N_BLKS:
            copy_blk(b + NBUF).start()
        if b == HEMI_BLKS - 1 or b == N_BLKS - 1:
            hemi = jnp.concatenate(dparts, axis=0)
            dparts = []
            s, mx = _topk_sum_and_max(hemi)
            heads.append((jnp.sum(hemi), mx, s))

    inv_n = jnp.float32(1.0 / N_HEMI)
    inv_k = jnp.float32(1.0 / TOPK)
    (suma, mxa, topa), (sumb, mxb, topb) = heads
    out = (suma * inv_n * wc_ref[0, 0] + mxa * wc_ref[0, 1]
           + topa * inv_k * wc_ref[0, 2] + sumb * inv_n * wc_ref[0, 3]
           + mxb * wc_ref[0, 4] + topb * inv_k * wc_ref[0, 5]
           + bc_ref[0, 0])
    out_ref[...] = out.reshape(1, 1)


@jax.jit
def kernel(x_fc, adj, l_loc, W1, Wl, W2, b2, Wc, bc):
    out = pl.pallas_call(
        _fused_kernel,
        in_specs=[
            pl.BlockSpec(memory_space=pl.ANY),
            pl.BlockSpec(memory_space=pl.ANY),
            pl.BlockSpec((N, DL), lambda: (0, 0)),
            pl.BlockSpec((D, H), lambda: (0, 0)),
            pl.BlockSpec((DL, H), lambda: (0, 0)),
            pl.BlockSpec((H, 2), lambda: (0, 0)),
            pl.BlockSpec((1, 2), lambda: (0, 0)),
            pl.BlockSpec((1, 6), lambda: (0, 0)),
            pl.BlockSpec((1, 1), lambda: (0, 0)),
        ],
        out_specs=pl.BlockSpec((1, 1), lambda: (0, 0)),
        out_shape=jax.ShapeDtypeStruct((1, 1), jnp.float32),
        scratch_shapes=[
            pltpu.VMEM((N, D), jnp.float32),
            pltpu.VMEM((NBUF, MAX_ROWS, N), jnp.float32),
            pltpu.SemaphoreType.DMA((NBUF,)),
            pltpu.SemaphoreType.DMA,
        ],
    )(x_fc, adj, l_loc, W1, Wl, W2, b2.reshape(1, 2), Wc, bc.reshape(1, 1))
    return out.reshape(-1)
